# baseline (speedup 1.0000x reference)
import dataclasses
import functools

import jax
import jax.numpy as jnp
from jax import lax
from jax.experimental import pallas as pl
from jax.experimental.pallas import tpu as pltpu
from jax.experimental.pallas import tpu_sc as plsc

N = 10000
D = 128
DH = D // 2
E = 320000
NC = 2
NS = 16
EPW = E // NS
K = 128
NCH = -(-EPW // K)
EPW_PAD = NCH * K
VALID_LAST = EPW - (NCH - 1) * K
N_PAD = 10112
STRIPE = N_PAD // NS
NB = 3


def _pre_body(h_ref, w_ref, av_ref, xlh_ref, as_ref, ad_ref, c_ref):
    xl = jnp.dot(h_ref[...], w_ref[...], preferred_element_type=jnp.float32)
    xlh_ref[0] = xl[:, :DH]
    xlh_ref[1] = xl[:, DH:]
    av = av_ref[...]
    a_s = jnp.sum(xl * av[0:1, :], axis=1, keepdims=True)
    a_d = jnp.sum(xl * av[1:2, :], axis=1, keepdims=True)
    as_ref[...] = a_s
    ad_ref[...] = a_d
    c = jnp.maximum(jnp.max(a_s) + jnp.max(a_d), 0.0)
    c_ref[...] = jnp.full((1, 16), c, jnp.float32)


def _pre(h, w, av):
    return pl.pallas_call(
        _pre_body,
        out_shape=(
            jax.ShapeDtypeStruct((NC, N, DH), jnp.float32),
            jax.ShapeDtypeStruct((N, 1), jnp.float32),
            jax.ShapeDtypeStruct((N, 1), jnp.float32),
            jax.ShapeDtypeStruct((1, 16), jnp.float32),
        ),
    )(h, w, av)


def _finish_layer(acc_ref, den_ref, as_ref, ad_ref, c_ref, xlh_ref, b_ref):
    v = as_ref[...] + ad_ref[...]
    e = jnp.maximum(v, 0.2 * v)
    ees = jnp.exp(e - c_ref[0:1, 0:1])
    xl = jnp.concatenate([xlh_ref[0], xlh_ref[1]], axis=1)
    num = jnp.concatenate([acc_ref[0], acc_ref[1]], axis=1) + ees * xl
    den = den_ref[...] + ees
    return num / (den + 1e-16) + b_ref[...]


def _eps_body(acc_ref, den_ref, as_ref, ad_ref, c_ref, xlh_ref, b_ref, o_ref):
    o_ref[...] = _finish_layer(acc_ref, den_ref, as_ref, ad_ref, c_ref,
                               xlh_ref, b_ref)


def _eps(acc, den, a_s, a_d, c, xlh, b):
    return pl.pallas_call(
        _eps_body,
        out_shape=jax.ShapeDtypeStruct((N, D), jnp.float32),
    )(acc, den, a_s, a_d, c, xlh, b)


_MB = 2000
_MG = N // _MB


def _mid_body(acc_ref, den_ref, as_ref, ad_ref, c_ref, xlh_ref, b_ref,
              w_ref, av_ref, xlh2_ref, as2_ref, ad2_ref, c2_ref, mx_ref):
    i = pl.program_id(0)
    h = _finish_layer(acc_ref, den_ref, as_ref, ad_ref, c_ref, xlh_ref, b_ref)
    xl = jnp.dot(h, w_ref[...], preferred_element_type=jnp.float32)
    xlh2_ref[0] = xl[:, :DH]
    xlh2_ref[1] = xl[:, DH:]
    av = av_ref[...]
    a_s = jnp.sum(xl * av[0:1, :], axis=1, keepdims=True)
    a_d = jnp.sum(xl * av[1:2, :], axis=1, keepdims=True)
    as2_ref[...] = a_s
    ad2_ref[...] = a_d
    bs = jnp.max(a_s)
    bd = jnp.max(a_d)

    @pl.when(i == 0)
    def _():
        mx_ref[0] = bs
        mx_ref[1] = bd

    @pl.when(i > 0)
    def _():
        mx_ref[0] = jnp.maximum(mx_ref[0], bs)
        mx_ref[1] = jnp.maximum(mx_ref[1], bd)

    @pl.when(i == _MG - 1)
    def _():
        c2_ref[...] = jnp.full(
            (1, 16), jnp.maximum(mx_ref[0] + mx_ref[1], 0.0), jnp.float32)


def _mid(acc, den, a_s, a_d, c, xlh, b, w, av):
    col = lambda i: (i, 0)
    full2 = lambda i: (0, 0)
    blk3 = lambda i: (0, i, 0)
    return pl.pallas_call(
        _mid_body,
        grid=(_MG,),
        in_specs=[
            pl.BlockSpec((NC, _MB, DH), blk3),
            pl.BlockSpec((_MB, 1), col),
            pl.BlockSpec((_MB, 1), col),
            pl.BlockSpec((_MB, 1), col),
            pl.BlockSpec((1, 16), full2),
            pl.BlockSpec((NC, _MB, DH), blk3),
            pl.BlockSpec((1, D), full2),
            pl.BlockSpec((D, D), full2),
            pl.BlockSpec((2, D), full2),
        ],
        out_specs=(
            pl.BlockSpec((NC, _MB, DH), blk3),
            pl.BlockSpec((_MB, 1), col),
            pl.BlockSpec((_MB, 1), col),
            pl.BlockSpec((1, 16), full2),
        ),
        out_shape=(
            jax.ShapeDtypeStruct((NC, N, DH), jnp.float32),
            jax.ShapeDtypeStruct((N, 1), jnp.float32),
            jax.ShapeDtypeStruct((N, 1), jnp.float32),
            jax.ShapeDtypeStruct((1, 16), jnp.float32),
        ),
        scratch_shapes=[pltpu.SMEM((2,), jnp.float32)],
    )(acc, den, a_s, a_d, c, xlh, b, w, av)


_mesh = plsc.VectorSubcoreMesh(core_axis_name="c", subcore_axis_name="s")

_sc_params = pltpu.CompilerParams()
for _field, _val in (("needs_layout_passes", False),
                     ("use_tc_tiling_on_sc", False)):
    if _field in pltpu.CompilerParams.__dataclass_fields__:
        _sc_params = dataclasses.replace(_sc_params, **{_field: _val})


@functools.partial(
    pl.kernel,
    out_type=(
        jax.ShapeDtypeStruct((NC, N_PAD, DH), jnp.float32),
        jax.ShapeDtypeStruct((NC, N_PAD), jnp.float32),
    ),
    mesh=_mesh,
    compiler_params=_sc_params,
    scratch_types=[
        pltpu.VMEM((N,), jnp.float32),
        pltpu.VMEM((N,), jnp.float32),
        pltpu.VMEM((EPW_PAD,), jnp.int32),
        pltpu.VMEM((EPW_PAD,), jnp.int32),
        pltpu.VMEM((NB * K,), jnp.float32),
        pltpu.VMEM((NB, K, DH), jnp.float32),
        pltpu.VMEM((16,), jnp.float32),
        pltpu.VMEM_SHARED((N_PAD, DH), jnp.float32),
        pltpu.VMEM_SHARED((N_PAD,), jnp.float32),
        pltpu.SemaphoreType.DMA,
        pltpu.SemaphoreType.DMA,
        pltpu.SemaphoreType.DMA,
        pltpu.SemaphoreType.DMA,
        pltpu.SemaphoreType.DMA,
        pltpu.SemaphoreType.DMA,
        pltpu.SemaphoreType.DMA,
        pltpu.SemaphoreType.DMA,
        pltpu.SemaphoreType.DMA,
    ],
)
def _sc_edges(xl2_hbm, src_hbm, dst_hbm, as_hbm, ad_hbm, cv_hbm,
              acc_out, den_out,
              as_v, ad_v, src_v, dst_v, ee_v, rows_v, c_v,
              acc_sh, den_sh,
              gsem0, gsem1, gsem2, rsem0, rsem1, rsem2,
              esem0, esem1, esem2):
    cid = lax.axis_index("c")
    sid = lax.axis_index("s")

    gsem = (gsem0, gsem1, gsem2)
    rsem = (rsem0, rsem1, rsem2)
    esem = (esem0, esem1, esem2)

    pltpu.sync_copy(as_hbm, as_v)
    pltpu.sync_copy(ad_hbm, ad_v)
    pltpu.sync_copy(src_hbm.at[sid], src_v.at[pl.ds(0, EPW)])
    pltpu.sync_copy(dst_hbm.at[sid], dst_v.at[pl.ds(0, EPW)])
    pltpu.sync_copy(cv_hbm, c_v)

    zero16 = jnp.zeros((16,), jnp.float32)

    lane16 = jax.lax.broadcasted_iota(jnp.int32, (16,), 0)
    for t in range((EPW_PAD - EPW) // 16):
        dst_v[pl.ds(EPW + t * 16, 16)] = (
            sid * ((EPW_PAD - EPW)) + t * 16 + lane16)

    off16 = jnp.full((16,), N, jnp.int32) * cid

    @pl.loop(0, EPW // 16)
    def _(i):
        sl = pl.ds(i * 16, 16)
        src_v[sl] = src_v[sl] + off16

    base_pad = jnp.full((16,), N, jnp.int32) * cid
    for t in range((EPW_PAD - EPW) // 16):
        src_v[pl.ds(EPW + t * 16, 16)] = base_pad

    @pl.loop(0, K)
    def _(i):
        for m in range(DH // 16):
            rows_v[0, i, pl.ds(m * 16, 16)] = zero16

    row_base = sid * STRIPE
    for j in range(4):
        pltpu.sync_copy(rows_v.at[0],
                        acc_sh.at[pl.ds(row_base + j * K, K)])
    pltpu.sync_copy(rows_v.at[0, pl.ds(0, STRIPE - 4 * K)],
                    acc_sh.at[pl.ds(row_base + 4 * K, STRIPE - 4 * K)])
    for j in range(9):
        pltpu.sync_copy(rows_v.at[0, 0],
                        den_sh.at[pl.ds(row_base + j * DH, DH)])
    pltpu.sync_copy(rows_v.at[0, 0, pl.ds(0, STRIPE - 9 * DH)],
                    den_sh.at[pl.ds(row_base + 9 * DH, STRIPE - 9 * DH)])

    plsc.subcore_barrier()

    def start_gather(ci, b):
        pltpu.async_copy(
            xl2_hbm.at[src_v.at[pl.ds(ci * K, K)]], rows_v.at[b], gsem[b])

    def wait_gather(ci, b):
        pltpu.make_async_copy(
            xl2_hbm.at[src_v.at[pl.ds(ci * K, K)]], rows_v.at[b],
            gsem[b]).wait()

    def compute_ee(ci, b, n_groups):
        for g in range(K // 16):
            off = b * K + g * 16
            if g < n_groups:
                s16 = src_v[pl.ds(ci * K + g * 16, 16)] - off16
                d16 = dst_v[pl.ds(ci * K + g * 16, 16)]
                asg = plsc.load_gather(as_v, [s16])
                adg = plsc.load_gather(ad_v, [d16])
                v = asg + adg
                e = jnp.maximum(v, 0.2 * v)
                ee_v[pl.ds(off, 16)] = jnp.exp(e - c_v[...])
            else:
                ee_v[pl.ds(off, 16)] = zero16

    def scale_rows(b):
        @pl.loop(0, K, step=8)
        def _(k0):
            for dk in range(8):
                k = k0 + dk
                esc = plsc.load_gather(
                    ee_v, [jnp.full((16,), b * K, jnp.int32) + k])
                for m in range(DH // 16):
                    sl = pl.ds(m * 16, 16)
                    rows_v[b, k, sl] = rows_v[b, k, sl] * esc

    def start_scatters(ci, b):
        dsl = dst_v.at[pl.ds(ci * K, K)]
        pltpu.async_copy(rows_v.at[b], acc_sh.at[dsl], rsem[b], add=True)
        pltpu.async_copy(ee_v.at[pl.ds(b * K, K)], den_sh.at[dsl],
                         esem[b], add=True)

    def wait_scatters(ci_prev, b):
        dsl = dst_v.at[pl.ds(ci_prev * K, K)]
        pltpu.make_async_copy(rows_v.at[b], acc_sh.at[dsl], rsem[b]).wait()
        pltpu.make_async_copy(ee_v.at[pl.ds(b * K, K)], den_sh.at[dsl],
                              esem[b]).wait()

    def warm(ci, b, n_groups):
        start_gather(ci, b)
        compute_ee(ci, b, n_groups)

    def fire(ci, b):
        wait_gather(ci, b)
        scale_rows(b)
        start_scatters(ci, b)

    def substep(t, first_warm=False, n_groups=K // 16):
        if t >= 2:
            fire(t - 2, (t - 2) % NB)
        if t < NCH:
            if not first_warm:
                wait_scatters(t - NB, t % NB)
            warm(t, t % NB, n_groups)

    substep(0, first_warm=True)
    substep(1, first_warm=True)
    substep(2, first_warm=True)

    @pl.loop(3, 156, step=3)
    def _(base):
        fire(base - 2, 1)
        wait_scatters(base - 3, 0)
        warm(base, 0, K // 16)
        fire(base - 1, 2)
        wait_scatters(base - 2, 1)
        warm(base + 1, 1, K // 16)
        fire(base, 0)
        wait_scatters(base - 1, 2)
        warm(base + 2, 2, K // 16)

    substep(156, n_groups=VALID_LAST // 16)
    substep(157)
    substep(158)

    wait_scatters(NCH - 3, (NCH - 3) % NB)
    wait_scatters(NCH - 2, (NCH - 2) % NB)
    wait_scatters(NCH - 1, (NCH - 1) % NB)

    plsc.subcore_barrier()

    pltpu.sync_copy(acc_sh.at[pl.ds(row_base, STRIPE)],
                    acc_out.at[cid, pl.ds(row_base, STRIPE)])

    @pl.when(sid == 0)
    def _():
        pltpu.sync_copy(den_sh, den_out.at[cid])


def kernel(x, edge_index, W1, a_src1, a_dst1, b1, W2, a_src2, a_dst2, b2):
    ei = edge_index.astype(jnp.int32)
    src = ei[0].reshape(NS, EPW)
    dst = ei[1].reshape(NS, EPW)
    av1 = jnp.stack([a_src1, a_dst1], axis=0)
    av2 = jnp.stack([a_src2, a_dst2], axis=0)

    xlh1, as1, ad1, c1 = _pre(x, W1, av1)
    acc1, den1 = _sc_edges(xlh1.reshape(NC * N, DH), src, dst,
                           as1.reshape(N), ad1.reshape(N), c1.reshape(16))
    h1 = _eps(acc1[:, :N, :], den1[0, :N].reshape(N, 1),
              as1, ad1, c1, xlh1, b1.reshape(1, D))
    xlh2, as2, ad2, c2 = _pre(h1, W2, av2)
    acc2, den2 = _sc_edges(xlh2.reshape(NC * N, DH), src, dst,
                           as2.reshape(N), ad2.reshape(N), c2.reshape(16))
    return _eps(acc2[:, :N, :], den2[0, :N].reshape(N, 1),
                as2, ad2, c2, xlh2, b2.reshape(1, D))


# speedup vs baseline: 1.0172x; 1.0172x over previous
import dataclasses
import functools

import jax
import jax.numpy as jnp
from jax import lax
from jax.experimental import pallas as pl
from jax.experimental.pallas import tpu as pltpu
from jax.experimental.pallas import tpu_sc as plsc

N = 10000
D = 128
DH = D // 2
E = 320000
NC = 2
NS = 16
EPW = E // NS
K = 128
NCH = -(-EPW // K)
EPW_PAD = NCH * K
VALID_LAST = EPW - (NCH - 1) * K
N_PAD = 10112
STRIPE = N_PAD // NS
NB = 3


def _pre_body(h_ref, w_ref, av_ref, xlh_ref, as_ref, ad_ref, c_ref):
    xl = jnp.dot(h_ref[...], w_ref[...], preferred_element_type=jnp.float32)
    xlh_ref[0] = xl[:, :DH]
    xlh_ref[1] = xl[:, DH:]
    av = av_ref[...]
    a_s = jnp.sum(xl * av[0:1, :], axis=1, keepdims=True)
    a_d = jnp.sum(xl * av[1:2, :], axis=1, keepdims=True)
    as_ref[...] = a_s
    ad_ref[...] = a_d
    c = jnp.maximum(jnp.max(a_s) + jnp.max(a_d), 0.0)
    c_ref[...] = jnp.full((1, 16), c, jnp.float32)


def _pre(h, w, av):
    return pl.pallas_call(
        _pre_body,
        out_shape=(
            jax.ShapeDtypeStruct((NC, N, DH), jnp.float32),
            jax.ShapeDtypeStruct((N, 1), jnp.float32),
            jax.ShapeDtypeStruct((N, 1), jnp.float32),
            jax.ShapeDtypeStruct((1, 16), jnp.float32),
        ),
    )(h, w, av)


def _finish_layer(acc_ref, den_ref, as_ref, ad_ref, c_ref, xlh_ref, b_ref):
    v = as_ref[...] + ad_ref[...]
    e = jnp.maximum(v, 0.2 * v)
    ees = jnp.exp(e - c_ref[0:1, 0:1])
    xl = jnp.concatenate([xlh_ref[0], xlh_ref[1]], axis=1)
    num = jnp.concatenate([acc_ref[0], acc_ref[1]], axis=1) + ees * xl
    den = den_ref[...] + ees
    return num / (den + 1e-16) + b_ref[...]


def _eps_body(acc_ref, den_ref, as_ref, ad_ref, c_ref, xlh_ref, b_ref, o_ref):
    o_ref[...] = _finish_layer(acc_ref, den_ref, as_ref, ad_ref, c_ref,
                               xlh_ref, b_ref)


def _eps(acc, den, a_s, a_d, c, xlh, b):
    return pl.pallas_call(
        _eps_body,
        out_shape=jax.ShapeDtypeStruct((N, D), jnp.float32),
    )(acc, den, a_s, a_d, c, xlh, b)


_MB = 2000
_MG = N // _MB


def _mid_body(acc_ref, den_ref, as_ref, ad_ref, c_ref, xlh_ref, b_ref,
              w_ref, av_ref, xlh2_ref, as2_ref, ad2_ref, c2_ref, mx_ref):
    i = pl.program_id(0)
    h = _finish_layer(acc_ref, den_ref, as_ref, ad_ref, c_ref, xlh_ref, b_ref)
    xl = jnp.dot(h, w_ref[...], preferred_element_type=jnp.float32)
    xlh2_ref[0] = xl[:, :DH]
    xlh2_ref[1] = xl[:, DH:]
    av = av_ref[...]
    a_s = jnp.sum(xl * av[0:1, :], axis=1, keepdims=True)
    a_d = jnp.sum(xl * av[1:2, :], axis=1, keepdims=True)
    as2_ref[...] = a_s
    ad2_ref[...] = a_d
    bs = jnp.max(a_s)
    bd = jnp.max(a_d)

    @pl.when(i == 0)
    def _():
        mx_ref[0] = bs
        mx_ref[1] = bd

    @pl.when(i > 0)
    def _():
        mx_ref[0] = jnp.maximum(mx_ref[0], bs)
        mx_ref[1] = jnp.maximum(mx_ref[1], bd)

    @pl.when(i == _MG - 1)
    def _():
        c2_ref[...] = jnp.full(
            (1, 16), jnp.maximum(mx_ref[0] + mx_ref[1], 0.0), jnp.float32)


def _mid(acc, den, a_s, a_d, c, xlh, b, w, av):
    col = lambda i: (i, 0)
    full2 = lambda i: (0, 0)
    blk3 = lambda i: (0, i, 0)
    return pl.pallas_call(
        _mid_body,
        grid=(_MG,),
        in_specs=[
            pl.BlockSpec((NC, _MB, DH), blk3),
            pl.BlockSpec((_MB, 1), col),
            pl.BlockSpec((_MB, 1), col),
            pl.BlockSpec((_MB, 1), col),
            pl.BlockSpec((1, 16), full2),
            pl.BlockSpec((NC, _MB, DH), blk3),
            pl.BlockSpec((1, D), full2),
            pl.BlockSpec((D, D), full2),
            pl.BlockSpec((2, D), full2),
        ],
        out_specs=(
            pl.BlockSpec((NC, _MB, DH), blk3),
            pl.BlockSpec((_MB, 1), col),
            pl.BlockSpec((_MB, 1), col),
            pl.BlockSpec((1, 16), full2),
        ),
        out_shape=(
            jax.ShapeDtypeStruct((NC, N, DH), jnp.float32),
            jax.ShapeDtypeStruct((N, 1), jnp.float32),
            jax.ShapeDtypeStruct((N, 1), jnp.float32),
            jax.ShapeDtypeStruct((1, 16), jnp.float32),
        ),
        scratch_shapes=[pltpu.SMEM((2,), jnp.float32)],
    )(acc, den, a_s, a_d, c, xlh, b, w, av)


_mesh = plsc.VectorSubcoreMesh(core_axis_name="c", subcore_axis_name="s")

_sc_params = pltpu.CompilerParams()
for _field, _val in (("needs_layout_passes", False),
                     ("use_tc_tiling_on_sc", False)):
    if _field in pltpu.CompilerParams.__dataclass_fields__:
        _sc_params = dataclasses.replace(_sc_params, **{_field: _val})


@functools.partial(
    pl.kernel,
    out_type=(
        jax.ShapeDtypeStruct((NC, N_PAD, DH), jnp.float32),
        jax.ShapeDtypeStruct((NC, N_PAD), jnp.float32),
    ),
    mesh=_mesh,
    compiler_params=_sc_params,
    scratch_types=[
        pltpu.VMEM((N,), jnp.float32),
        pltpu.VMEM((N,), jnp.float32),
        pltpu.VMEM((EPW_PAD,), jnp.int32),
        pltpu.VMEM((EPW_PAD,), jnp.int32),
        pltpu.VMEM((NB * K,), jnp.float32),
        pltpu.VMEM((NB, K, DH), jnp.float32),
        pltpu.VMEM((16,), jnp.float32),
        pltpu.VMEM_SHARED((N_PAD, DH), jnp.float32),
        pltpu.VMEM_SHARED((N_PAD,), jnp.float32),
        pltpu.SemaphoreType.DMA,
        pltpu.SemaphoreType.DMA,
        pltpu.SemaphoreType.DMA,
        pltpu.SemaphoreType.DMA,
        pltpu.SemaphoreType.DMA,
        pltpu.SemaphoreType.DMA,
        pltpu.SemaphoreType.DMA,
        pltpu.SemaphoreType.DMA,
        pltpu.SemaphoreType.DMA,
    ],
)
def _sc_edges(xl2_hbm, src_hbm, dst_hbm, as_hbm, ad_hbm, cv_hbm,
              acc_out, den_out,
              as_v, ad_v, src_v, dst_v, ee_v, rows_v, c_v,
              acc_sh, den_sh,
              gsem0, gsem1, gsem2, rsem0, rsem1, rsem2,
              esem0, esem1, esem2):
    cid = lax.axis_index("c")
    sid = lax.axis_index("s")

    gsem = (gsem0, gsem1, gsem2)
    rsem = (rsem0, rsem1, rsem2)
    esem = (esem0, esem1, esem2)

    pltpu.sync_copy(as_hbm, as_v)
    pltpu.sync_copy(ad_hbm, ad_v)
    pltpu.sync_copy(src_hbm.at[sid], src_v.at[pl.ds(0, EPW)])
    pltpu.sync_copy(dst_hbm.at[sid], dst_v.at[pl.ds(0, EPW)])
    pltpu.sync_copy(cv_hbm, c_v)

    zero16 = jnp.zeros((16,), jnp.float32)

    lane16 = jax.lax.broadcasted_iota(jnp.int32, (16,), 0)
    for t in range((EPW_PAD - EPW) // 16):
        dst_v[pl.ds(EPW + t * 16, 16)] = (
            sid * ((EPW_PAD - EPW)) + t * 16 + lane16)

    off16 = jnp.full((16,), N, jnp.int32) * cid

    @pl.loop(0, EPW // 16)
    def _(i):
        sl = pl.ds(i * 16, 16)
        src_v[sl] = src_v[sl] + off16

    base_pad = jnp.full((16,), N, jnp.int32) * cid
    for t in range((EPW_PAD - EPW) // 16):
        src_v[pl.ds(EPW + t * 16, 16)] = base_pad

    @pl.loop(0, K)
    def _(i):
        for m in range(DH // 16):
            rows_v[0, i, pl.ds(m * 16, 16)] = zero16

    row_base = sid * STRIPE
    for j in range(4):
        pltpu.sync_copy(rows_v.at[0],
                        acc_sh.at[pl.ds(row_base + j * K, K)])
    pltpu.sync_copy(rows_v.at[0, pl.ds(0, STRIPE - 4 * K)],
                    acc_sh.at[pl.ds(row_base + 4 * K, STRIPE - 4 * K)])
    for j in range(9):
        pltpu.sync_copy(rows_v.at[0, 0],
                        den_sh.at[pl.ds(row_base + j * DH, DH)])
    pltpu.sync_copy(rows_v.at[0, 0, pl.ds(0, STRIPE - 9 * DH)],
                    den_sh.at[pl.ds(row_base + 9 * DH, STRIPE - 9 * DH)])

    plsc.subcore_barrier()

    def start_gather(ci, b):
        pltpu.async_copy(
            xl2_hbm.at[src_v.at[pl.ds(ci * K, K)]], rows_v.at[b], gsem[b])

    def wait_gather(ci, b):
        pltpu.make_async_copy(
            xl2_hbm.at[src_v.at[pl.ds(ci * K, K)]], rows_v.at[b],
            gsem[b]).wait()

    def compute_ee(ci, b, n_groups):
        for g in range(K // 16):
            off = b * K + g * 16
            if g < n_groups:
                s16 = src_v[pl.ds(ci * K + g * 16, 16)] - off16
                d16 = dst_v[pl.ds(ci * K + g * 16, 16)]
                asg = plsc.load_gather(as_v, [s16])
                adg = plsc.load_gather(ad_v, [d16])
                v = asg + adg
                e = jnp.maximum(v, 0.2 * v)
                ee_v[pl.ds(off, 16)] = jnp.exp(e - c_v[...])
            else:
                ee_v[pl.ds(off, 16)] = zero16

    def scale_rows(b):
        @pl.loop(0, K, step=4)
        def _(k0):
            for dk in range(4):
                k = k0 + dk
                esc = plsc.load_gather(
                    ee_v, [jnp.full((16,), b * K, jnp.int32) + k])
                for m in range(DH // 16):
                    sl = pl.ds(m * 16, 16)
                    rows_v[b, k, sl] = rows_v[b, k, sl] * esc

    def start_scatters(ci, b):
        dsl = dst_v.at[pl.ds(ci * K, K)]
        pltpu.async_copy(rows_v.at[b], acc_sh.at[dsl], rsem[b], add=True)
        pltpu.async_copy(ee_v.at[pl.ds(b * K, K)], den_sh.at[dsl],
                         esem[b], add=True)

    def wait_scatters(ci_prev, b):
        dsl = dst_v.at[pl.ds(ci_prev * K, K)]
        pltpu.make_async_copy(rows_v.at[b], acc_sh.at[dsl], rsem[b]).wait()
        pltpu.make_async_copy(ee_v.at[pl.ds(b * K, K)], den_sh.at[dsl],
                              esem[b]).wait()

    def warm(ci, b, n_groups):
        start_gather(ci, b)
        compute_ee(ci, b, n_groups)

    def fire(ci, b):
        wait_gather(ci, b)
        scale_rows(b)
        start_scatters(ci, b)

    def substep(t, first_warm=False, n_groups=K // 16):
        if t >= 2:
            fire(t - 2, (t - 2) % NB)
        if t < NCH:
            if not first_warm:
                wait_scatters(t - NB, t % NB)
            warm(t, t % NB, n_groups)

    substep(0, first_warm=True)
    substep(1, first_warm=True)
    substep(2, first_warm=True)

    @pl.loop(3, 156, step=3)
    def _(base):
        fire(base - 2, 1)
        wait_scatters(base - 3, 0)
        warm(base, 0, K // 16)
        fire(base - 1, 2)
        wait_scatters(base - 2, 1)
        warm(base + 1, 1, K // 16)
        fire(base, 0)
        wait_scatters(base - 1, 2)
        warm(base + 2, 2, K // 16)

    substep(156, n_groups=VALID_LAST // 16)
    substep(157)
    substep(158)

    wait_scatters(NCH - 3, (NCH - 3) % NB)
    wait_scatters(NCH - 2, (NCH - 2) % NB)
    wait_scatters(NCH - 1, (NCH - 1) % NB)

    plsc.subcore_barrier()

    pltpu.sync_copy(acc_sh.at[pl.ds(row_base, STRIPE)],
                    acc_out.at[cid, pl.ds(row_base, STRIPE)])

    @pl.when(sid == 0)
    def _():
        pltpu.sync_copy(den_sh, den_out.at[cid])


def kernel(x, edge_index, W1, a_src1, a_dst1, b1, W2, a_src2, a_dst2, b2):
    ei = edge_index.astype(jnp.int32)
    src = ei[0].reshape(NS, EPW)
    dst = ei[1].reshape(NS, EPW)
    av1 = jnp.stack([a_src1, a_dst1], axis=0)
    av2 = jnp.stack([a_src2, a_dst2], axis=0)

    xlh1, as1, ad1, c1 = _pre(x, W1, av1)
    acc1, den1 = _sc_edges(xlh1.reshape(NC * N, DH), src, dst,
                           as1.reshape(N), ad1.reshape(N), c1.reshape(16))
    h1 = _eps(acc1[:, :N, :], den1[0, :N].reshape(N, 1),
              as1, ad1, c1, xlh1, b1.reshape(1, D))
    xlh2, as2, ad2, c2 = _pre(h1, W2, av2)
    acc2, den2 = _sc_edges(xlh2.reshape(NC * N, DH), src, dst,
                           as2.reshape(N), ad2.reshape(N), c2.reshape(16))
    return _eps(acc2[:, :N, :], den2[0, :N].reshape(N, 1),
                as2, ad2, c2, xlh2, b2.reshape(1, D))


# speedup vs baseline: 1.1469x; 1.1275x over previous
import dataclasses
import functools

import jax
import jax.numpy as jnp
from jax import lax
from jax.experimental import pallas as pl
from jax.experimental.pallas import tpu as pltpu
from jax.experimental.pallas import tpu_sc as plsc

N = 10000
D = 128
DH = D // 2
E = 320000
NC = 2
NS = 16
EPW = E // NS
K = 128
NCH = -(-EPW // K)
EPW_PAD = NCH * K
VALID_LAST = EPW - (NCH - 1) * K
N_PAD = 10112
STRIPE = N_PAD // NS
NB = 3


def _pre_body(h_ref, w_ref, av_ref, xlh_ref, as_ref, ad_ref, c_ref):
    xl = jnp.dot(h_ref[...], w_ref[...], preferred_element_type=jnp.float32)
    xlh_ref[0] = xl[:, :DH]
    xlh_ref[1] = xl[:, DH:]
    av = av_ref[...]
    a_s = jnp.sum(xl * av[0:1, :], axis=1, keepdims=True)
    a_d = jnp.sum(xl * av[1:2, :], axis=1, keepdims=True)
    as_ref[...] = a_s
    ad_ref[...] = a_d
    c = jnp.maximum(jnp.max(a_s) + jnp.max(a_d), 0.0)
    c_ref[...] = jnp.full((1, 16), c, jnp.float32)


def _pre(h, w, av):
    return pl.pallas_call(
        _pre_body,
        out_shape=(
            jax.ShapeDtypeStruct((NC, N, DH), jnp.float32),
            jax.ShapeDtypeStruct((N, 1), jnp.float32),
            jax.ShapeDtypeStruct((N, 1), jnp.float32),
            jax.ShapeDtypeStruct((1, 16), jnp.float32),
        ),
    )(h, w, av)


def _finish_layer(acc_ref, den_ref, as_ref, ad_ref, c_ref, xlh_ref, b_ref):
    v = as_ref[...] + ad_ref[...]
    e = jnp.maximum(v, 0.2 * v)
    ees = jnp.exp(e - c_ref[0:1, 0:1])
    xl = jnp.concatenate([xlh_ref[0], xlh_ref[1]], axis=1)
    num = jnp.concatenate([acc_ref[0], acc_ref[1]], axis=1) + ees * xl
    den = den_ref[...] + ees
    return num / (den + 1e-16) + b_ref[...]


def _eps_body(acc_ref, den_ref, as_ref, ad_ref, c_ref, xlh_ref, b_ref, o_ref):
    o_ref[...] = _finish_layer(acc_ref, den_ref, as_ref, ad_ref, c_ref,
                               xlh_ref, b_ref)


def _eps(acc, den, a_s, a_d, c, xlh, b):
    return pl.pallas_call(
        _eps_body,
        out_shape=jax.ShapeDtypeStruct((N, D), jnp.float32),
    )(acc, den, a_s, a_d, c, xlh, b)


_MB = 2000
_MG = N // _MB


def _mid_body(acc_ref, den_ref, as_ref, ad_ref, c_ref, xlh_ref, b_ref,
              w_ref, av_ref, xlh2_ref, as2_ref, ad2_ref, c2_ref, mx_ref):
    i = pl.program_id(0)
    h = _finish_layer(acc_ref, den_ref, as_ref, ad_ref, c_ref, xlh_ref, b_ref)
    xl = jnp.dot(h, w_ref[...], preferred_element_type=jnp.float32)
    xlh2_ref[0] = xl[:, :DH]
    xlh2_ref[1] = xl[:, DH:]
    av = av_ref[...]
    a_s = jnp.sum(xl * av[0:1, :], axis=1, keepdims=True)
    a_d = jnp.sum(xl * av[1:2, :], axis=1, keepdims=True)
    as2_ref[...] = a_s
    ad2_ref[...] = a_d
    bs = jnp.max(a_s)
    bd = jnp.max(a_d)

    @pl.when(i == 0)
    def _():
        mx_ref[0] = bs
        mx_ref[1] = bd

    @pl.when(i > 0)
    def _():
        mx_ref[0] = jnp.maximum(mx_ref[0], bs)
        mx_ref[1] = jnp.maximum(mx_ref[1], bd)

    @pl.when(i == _MG - 1)
    def _():
        c2_ref[...] = jnp.full(
            (1, 16), jnp.maximum(mx_ref[0] + mx_ref[1], 0.0), jnp.float32)


def _mid(acc, den, a_s, a_d, c, xlh, b, w, av):
    col = lambda i: (i, 0)
    full2 = lambda i: (0, 0)
    blk3 = lambda i: (0, i, 0)
    return pl.pallas_call(
        _mid_body,
        grid=(_MG,),
        in_specs=[
            pl.BlockSpec((NC, _MB, DH), blk3),
            pl.BlockSpec((_MB, 1), col),
            pl.BlockSpec((_MB, 1), col),
            pl.BlockSpec((_MB, 1), col),
            pl.BlockSpec((1, 16), full2),
            pl.BlockSpec((NC, _MB, DH), blk3),
            pl.BlockSpec((1, D), full2),
            pl.BlockSpec((D, D), full2),
            pl.BlockSpec((2, D), full2),
        ],
        out_specs=(
            pl.BlockSpec((NC, _MB, DH), blk3),
            pl.BlockSpec((_MB, 1), col),
            pl.BlockSpec((_MB, 1), col),
            pl.BlockSpec((1, 16), full2),
        ),
        out_shape=(
            jax.ShapeDtypeStruct((NC, N, DH), jnp.float32),
            jax.ShapeDtypeStruct((N, 1), jnp.float32),
            jax.ShapeDtypeStruct((N, 1), jnp.float32),
            jax.ShapeDtypeStruct((1, 16), jnp.float32),
        ),
        scratch_shapes=[pltpu.SMEM((2,), jnp.float32)],
    )(acc, den, a_s, a_d, c, xlh, b, w, av)


_mesh = plsc.VectorSubcoreMesh(core_axis_name="c", subcore_axis_name="s")

_sc_params = pltpu.CompilerParams()
for _field, _val in (("needs_layout_passes", False),
                     ("use_tc_tiling_on_sc", False)):
    if _field in pltpu.CompilerParams.__dataclass_fields__:
        _sc_params = dataclasses.replace(_sc_params, **{_field: _val})


@functools.partial(
    pl.kernel,
    out_type=(
        jax.ShapeDtypeStruct((NC, N_PAD, DH), jnp.float32),
        jax.ShapeDtypeStruct((NC, N_PAD), jnp.float32),
    ),
    mesh=_mesh,
    compiler_params=_sc_params,
    scratch_types=[
        pltpu.VMEM((N,), jnp.float32),
        pltpu.VMEM((N,), jnp.float32),
        pltpu.VMEM((EPW_PAD,), jnp.int32),
        pltpu.VMEM((NCH, K), jnp.int32),
        pltpu.VMEM((NB * K,), jnp.float32),
        pltpu.VMEM((NB, K, DH), jnp.float32),
        pltpu.VMEM((16,), jnp.float32),
        pltpu.VMEM_SHARED((N_PAD, DH), jnp.float32),
        pltpu.VMEM_SHARED((N_PAD,), jnp.float32),
        pltpu.SemaphoreType.DMA,
        pltpu.SemaphoreType.DMA,
        pltpu.SemaphoreType.DMA,
        pltpu.SemaphoreType.DMA,
        pltpu.SemaphoreType.DMA,
        pltpu.SemaphoreType.DMA,
        pltpu.SemaphoreType.DMA,
        pltpu.SemaphoreType.DMA,
        pltpu.SemaphoreType.DMA,
    ],
)
def _sc_edges(xl2_hbm, src_hbm, dst_hbm, as_hbm, ad_hbm, cv_hbm,
              acc_out, den_out,
              as_v, ad_v, src_v, dst_v, ee_v, rows_v, c_v,
              acc_sh, den_sh,
              gsem0, gsem1, gsem2, rsem0, rsem1, rsem2,
              esem0, esem1, esem2):
    cid = lax.axis_index("c")
    sid = lax.axis_index("s")

    gsem = (gsem0, gsem1, gsem2)
    rsem = (rsem0, rsem1, rsem2)
    esem = (esem0, esem1, esem2)

    pltpu.sync_copy(as_hbm, as_v)
    pltpu.sync_copy(ad_hbm, ad_v)
    pltpu.sync_copy(src_hbm.at[sid], src_v)
    pltpu.sync_copy(dst_hbm.at[sid], dst_v)
    pltpu.sync_copy(cv_hbm, c_v)

    zero16 = jnp.zeros((16,), jnp.float32)

    off16 = jnp.full((16,), N, jnp.int32) * cid

    @pl.loop(0, EPW_PAD // 16)
    def _(i):
        sl = pl.ds(i * 16, 16)
        src_v[sl] = src_v[sl] + off16

    @pl.loop(0, K)
    def _(i):
        for m in range(DH // 16):
            rows_v[0, i, pl.ds(m * 16, 16)] = zero16

    row_base = sid * STRIPE
    for j in range(4):
        pltpu.sync_copy(rows_v.at[0],
                        acc_sh.at[pl.ds(row_base + j * K, K)])
    pltpu.sync_copy(rows_v.at[0, pl.ds(0, STRIPE - 4 * K)],
                    acc_sh.at[pl.ds(row_base + 4 * K, STRIPE - 4 * K)])
    for j in range(9):
        pltpu.sync_copy(rows_v.at[0, 0],
                        den_sh.at[pl.ds(row_base + j * DH, DH)])
    pltpu.sync_copy(rows_v.at[0, 0, pl.ds(0, STRIPE - 9 * DH)],
                    den_sh.at[pl.ds(row_base + 9 * DH, STRIPE - 9 * DH)])

    plsc.subcore_barrier()

    def start_gather(ci, b):
        pltpu.async_copy(
            xl2_hbm.at[src_v.at[pl.ds(ci * K, K)]], rows_v.at[b], gsem[b])

    def wait_gather(ci, b):
        pltpu.make_async_copy(
            xl2_hbm.at[src_v.at[pl.ds(ci * K, K)]], rows_v.at[b],
            gsem[b]).wait()

    def compute_ee(ci, b, n_groups):
        for g in range(K // 16):
            off = b * K + g * 16
            if g < n_groups:
                s16 = src_v[pl.ds(ci * K + g * 16, 16)] - off16
                d16 = dst_v[ci, pl.ds(g * 16, 16)]
                asg = plsc.load_gather(as_v, [s16])
                adg = plsc.load_gather(ad_v, [d16])
                v = asg + adg
                e = jnp.maximum(v, 0.2 * v)
                ee_v[pl.ds(off, 16)] = jnp.exp(e - c_v[...])
            else:
                ee_v[pl.ds(off, 16)] = zero16

    def scale_rows(b):
        @pl.loop(0, K, step=4)
        def _(k0):
            for dk in range(4):
                k = k0 + dk
                esc = plsc.load_gather(
                    ee_v, [jnp.full((16,), b * K, jnp.int32) + k])
                for m in range(DH // 16):
                    sl = pl.ds(m * 16, 16)
                    rows_v[b, k, sl] = rows_v[b, k, sl] * esc

    def start_scatters(ci, b):
        pltpu.async_copy(rows_v.at[b], acc_sh.at[dst_v.at[ci]], rsem[b],
                         add=True)
        pltpu.async_copy(ee_v.at[pl.ds(b * K, K)], den_sh.at[dst_v.at[ci]],
                         esem[b], add=True)

    def wait_scatters(ci_prev, b):
        pltpu.make_async_copy(rows_v.at[b], acc_sh.at[dst_v.at[ci_prev]],
                              rsem[b]).wait()
        pltpu.make_async_copy(ee_v.at[pl.ds(b * K, K)],
                              den_sh.at[dst_v.at[ci_prev]], esem[b]).wait()

    def warm(ci, b, n_groups):
        start_gather(ci, b)
        compute_ee(ci, b, n_groups)

    def fire(ci, b):
        wait_gather(ci, b)
        scale_rows(b)
        start_scatters(ci, b)

    def substep(t, first_warm=False, n_groups=K // 16):
        if t >= 2:
            fire(t - 2, (t - 2) % NB)
        if t < NCH:
            if not first_warm:
                wait_scatters(t - NB, t % NB)
            warm(t, t % NB, n_groups)

    substep(0, first_warm=True)
    substep(1, first_warm=True)
    substep(2, first_warm=True)

    @pl.loop(3, 156, step=3)
    def _(base):
        fire(base - 2, 1)
        wait_scatters(base - 3, 0)
        warm(base, 0, K // 16)
        fire(base - 1, 2)
        wait_scatters(base - 2, 1)
        warm(base + 1, 1, K // 16)
        fire(base, 0)
        wait_scatters(base - 1, 2)
        warm(base + 2, 2, K // 16)

    substep(156, n_groups=VALID_LAST // 16)
    substep(157)
    substep(158)

    wait_scatters(NCH - 3, (NCH - 3) % NB)
    wait_scatters(NCH - 2, (NCH - 2) % NB)
    wait_scatters(NCH - 1, (NCH - 1) % NB)

    plsc.subcore_barrier()

    pltpu.sync_copy(acc_sh.at[pl.ds(row_base, STRIPE)],
                    acc_out.at[cid, pl.ds(row_base, STRIPE)])

    @pl.when(sid == 0)
    def _():
        pltpu.sync_copy(den_sh, den_out.at[cid])


def kernel(x, edge_index, W1, a_src1, a_dst1, b1, W2, a_src2, a_dst2, b2):
    ei = edge_index.astype(jnp.int32)
    src = ei[0].reshape(NS, EPW)
    dst = ei[1].reshape(NS, EPW)
    npad = EPW_PAD - EPW
    pad_s = (jnp.arange(NS * npad, dtype=jnp.int32) * 97 + 13) % N
    pad_d = (jnp.arange(NS * npad, dtype=jnp.int32) * 131 + 7) % N
    src_flat = jnp.concatenate([src, pad_s.reshape(NS, npad)], axis=1)
    dst_chunk = jnp.concatenate([dst, pad_d.reshape(NS, npad)],
                                axis=1).reshape(NS, NCH, K)
    av1 = jnp.stack([a_src1, a_dst1], axis=0)
    av2 = jnp.stack([a_src2, a_dst2], axis=0)

    xlh1, as1, ad1, c1 = _pre(x, W1, av1)
    acc1, den1 = _sc_edges(xlh1.reshape(NC * N, DH), src_flat, dst_chunk,
                           as1.reshape(N), ad1.reshape(N), c1.reshape(16))
    xlh2, as2, ad2, c2 = _mid(acc1[:, :N, :], den1[0, :N].reshape(N, 1),
                              as1, ad1, c1, xlh1, b1.reshape(1, D),
                              W2, av2)
    acc2, den2 = _sc_edges(xlh2.reshape(NC * N, DH), src_flat, dst_chunk,
                           as2.reshape(N), ad2.reshape(N), c2.reshape(16))
    return _eps(acc2[:, :N, :], den2[0, :N].reshape(N, 1),
                as2, ad2, c2, xlh2, b2.reshape(1, D))


# speedup vs baseline: 1.2314x; 1.0737x over previous
import dataclasses
import functools

import jax
import jax.numpy as jnp
from jax import lax
from jax.experimental import pallas as pl
from jax.experimental.pallas import tpu as pltpu
from jax.experimental.pallas import tpu_sc as plsc

N = 10000
D = 128
DH = D // 2
E = 320000
NC = 2
NS = 16
EPW = E // NS
K = 128
NCH = -(-EPW // K)
EPW_PAD = NCH * K
VALID_LAST = EPW - (NCH - 1) * K
N_PAD = 10112
STRIPE = N_PAD // NS
NB = 3


def _pre_body(h_ref, w_ref, av_ref, xlh_ref, as_ref, ad_ref, c_ref):
    xl = jnp.dot(h_ref[...], w_ref[...], preferred_element_type=jnp.float32)
    xlh_ref[0] = xl[:, :DH]
    xlh_ref[1] = xl[:, DH:]
    av = av_ref[...]
    a_s = jnp.sum(xl * av[0:1, :], axis=1, keepdims=True)
    a_d = jnp.sum(xl * av[1:2, :], axis=1, keepdims=True)
    as_ref[...] = a_s
    ad_ref[...] = a_d
    c = jnp.maximum(jnp.max(a_s) + jnp.max(a_d), 0.0)
    c_ref[...] = jnp.full((1, 16), c, jnp.float32)


def _pre(h, w, av):
    return pl.pallas_call(
        _pre_body,
        out_shape=(
            jax.ShapeDtypeStruct((NC, N, DH), jnp.float32),
            jax.ShapeDtypeStruct((N, 1), jnp.float32),
            jax.ShapeDtypeStruct((N, 1), jnp.float32),
            jax.ShapeDtypeStruct((1, 16), jnp.float32),
        ),
    )(h, w, av)


def _finish_layer(acc_ref, den_ref, as_ref, ad_ref, c_ref, xlh_ref, b_ref):
    v = as_ref[...] + ad_ref[...]
    e = jnp.maximum(v, 0.2 * v)
    ees = jnp.exp(e - c_ref[0:1, 0:1])
    xl = jnp.concatenate([xlh_ref[0], xlh_ref[1]], axis=1)
    num = acc_ref[...] + ees * xl
    den = den_ref[...] + ees
    return num / (den + 1e-16) + b_ref[...]


def _eps_body(acc_ref, den_ref, as_ref, ad_ref, c_ref, xlh_ref, b_ref, o_ref):
    o_ref[...] = _finish_layer(acc_ref, den_ref, as_ref, ad_ref, c_ref,
                               xlh_ref, b_ref)


def _eps(acc, den, a_s, a_d, c, xlh, b):
    col = lambda i: (i, 0)
    full2 = lambda i: (0, 0)
    blk3 = lambda i: (0, i, 0)
    return pl.pallas_call(
        _eps_body,
        grid=(N // _MB,),
        in_specs=[
            pl.BlockSpec((_MB, D), col),
            pl.BlockSpec((_MB, 1), col),
            pl.BlockSpec((_MB, 1), col),
            pl.BlockSpec((_MB, 1), col),
            pl.BlockSpec((1, 16), full2),
            pl.BlockSpec((NC, _MB, DH), blk3),
            pl.BlockSpec((1, D), full2),
        ],
        out_specs=pl.BlockSpec((_MB, D), col),
        out_shape=jax.ShapeDtypeStruct((N, D), jnp.float32),
    )(acc, den, a_s, a_d, c, xlh, b)


_MB = 2000
_MG = N // _MB


def _mid_body(acc_ref, den_ref, as_ref, ad_ref, c_ref, xlh_ref, b_ref,
              w_ref, av_ref, xlh2_ref, as2_ref, ad2_ref, c2_ref, mx_ref):
    i = pl.program_id(0)
    h = _finish_layer(acc_ref, den_ref, as_ref, ad_ref, c_ref, xlh_ref, b_ref)
    xl = jnp.dot(h, w_ref[...], preferred_element_type=jnp.float32)
    xlh2_ref[0] = xl[:, :DH]
    xlh2_ref[1] = xl[:, DH:]
    av = av_ref[...]
    a_s = jnp.sum(xl * av[0:1, :], axis=1, keepdims=True)
    a_d = jnp.sum(xl * av[1:2, :], axis=1, keepdims=True)
    as2_ref[...] = a_s
    ad2_ref[...] = a_d
    bs = jnp.max(a_s)
    bd = jnp.max(a_d)

    @pl.when(i == 0)
    def _():
        mx_ref[0] = bs
        mx_ref[1] = bd

    @pl.when(i > 0)
    def _():
        mx_ref[0] = jnp.maximum(mx_ref[0], bs)
        mx_ref[1] = jnp.maximum(mx_ref[1], bd)

    @pl.when(i == _MG - 1)
    def _():
        c2_ref[...] = jnp.full(
            (1, 16), jnp.maximum(mx_ref[0] + mx_ref[1], 0.0), jnp.float32)


def _mid(acc, den, a_s, a_d, c, xlh, b, w, av):
    col = lambda i: (i, 0)
    full2 = lambda i: (0, 0)
    blk3 = lambda i: (0, i, 0)
    return pl.pallas_call(
        _mid_body,
        grid=(_MG,),
        in_specs=[
            pl.BlockSpec((_MB, D), col),
            pl.BlockSpec((_MB, 1), col),
            pl.BlockSpec((_MB, 1), col),
            pl.BlockSpec((_MB, 1), col),
            pl.BlockSpec((1, 16), full2),
            pl.BlockSpec((NC, _MB, DH), blk3),
            pl.BlockSpec((1, D), full2),
            pl.BlockSpec((D, D), full2),
            pl.BlockSpec((2, D), full2),
        ],
        out_specs=(
            pl.BlockSpec((NC, _MB, DH), blk3),
            pl.BlockSpec((_MB, 1), col),
            pl.BlockSpec((_MB, 1), col),
            pl.BlockSpec((1, 16), full2),
        ),
        out_shape=(
            jax.ShapeDtypeStruct((NC, N, DH), jnp.float32),
            jax.ShapeDtypeStruct((N, 1), jnp.float32),
            jax.ShapeDtypeStruct((N, 1), jnp.float32),
            jax.ShapeDtypeStruct((1, 16), jnp.float32),
        ),
        scratch_shapes=[pltpu.SMEM((2,), jnp.float32)],
    )(acc, den, a_s, a_d, c, xlh, b, w, av)


_mesh = plsc.VectorSubcoreMesh(core_axis_name="c", subcore_axis_name="s")

_sc_params = pltpu.CompilerParams()
for _field, _val in (("needs_layout_passes", False),
                     ("use_tc_tiling_on_sc", False)):
    if _field in pltpu.CompilerParams.__dataclass_fields__:
        _sc_params = dataclasses.replace(_sc_params, **{_field: _val})


@functools.partial(
    pl.kernel,
    out_type=(
        jax.ShapeDtypeStruct((N_PAD, D), jnp.float32),
        jax.ShapeDtypeStruct((NC, N_PAD), jnp.float32),
    ),
    mesh=_mesh,
    compiler_params=_sc_params,
    scratch_types=[
        pltpu.VMEM((N,), jnp.float32),
        pltpu.VMEM((N,), jnp.float32),
        pltpu.VMEM((EPW_PAD,), jnp.int32),
        pltpu.VMEM((NCH, K), jnp.int32),
        pltpu.VMEM((NB * K,), jnp.float32),
        pltpu.VMEM((NB, K, DH), jnp.float32),
        pltpu.VMEM((16,), jnp.float32),
        pltpu.VMEM_SHARED((N_PAD, DH), jnp.float32),
        pltpu.VMEM_SHARED((N_PAD,), jnp.float32),
        pltpu.SemaphoreType.DMA,
        pltpu.SemaphoreType.DMA,
        pltpu.SemaphoreType.DMA,
        pltpu.SemaphoreType.DMA,
        pltpu.SemaphoreType.DMA,
        pltpu.SemaphoreType.DMA,
        pltpu.SemaphoreType.DMA,
        pltpu.SemaphoreType.DMA,
        pltpu.SemaphoreType.DMA,
    ],
)
def _sc_edges(xl2_hbm, src_hbm, dst_hbm, as_hbm, ad_hbm, cv_hbm,
              acc_out, den_out,
              as_v, ad_v, src_v, dst_v, ee_v, rows_v, c_v,
              acc_sh, den_sh,
              gsem0, gsem1, gsem2, rsem0, rsem1, rsem2,
              esem0, esem1, esem2):
    cid = lax.axis_index("c")
    sid = lax.axis_index("s")

    gsem = (gsem0, gsem1, gsem2)
    rsem = (rsem0, rsem1, rsem2)
    esem = (esem0, esem1, esem2)

    st_as = pltpu.async_copy(as_hbm, as_v, gsem0)
    st_ad = pltpu.async_copy(ad_hbm, ad_v, gsem1)
    st_src = pltpu.async_copy(src_hbm.at[sid], src_v, gsem2)
    st_dst = pltpu.async_copy(dst_hbm.at[sid], dst_v, rsem0)
    st_c = pltpu.async_copy(cv_hbm, c_v, rsem1)

    zero16 = jnp.zeros((16,), jnp.float32)

    @pl.loop(0, K)
    def _(i):
        for m in range(DH // 16):
            rows_v[0, i, pl.ds(m * 16, 16)] = zero16

    off16 = jnp.full((16,), N, jnp.int32) * cid

    st_src.wait()

    @pl.loop(0, EPW_PAD // 16)
    def _(i):
        sl = pl.ds(i * 16, 16)
        src_v[sl] = src_v[sl] + off16

    st_as.wait()
    st_ad.wait()
    st_dst.wait()
    st_c.wait()

    row_base = sid * STRIPE
    for j in range(4):
        pltpu.sync_copy(rows_v.at[0],
                        acc_sh.at[pl.ds(row_base + j * K, K)])
    pltpu.sync_copy(rows_v.at[0, pl.ds(0, STRIPE - 4 * K)],
                    acc_sh.at[pl.ds(row_base + 4 * K, STRIPE - 4 * K)])
    for j in range(9):
        pltpu.sync_copy(rows_v.at[0, 0],
                        den_sh.at[pl.ds(row_base + j * DH, DH)])
    pltpu.sync_copy(rows_v.at[0, 0, pl.ds(0, STRIPE - 9 * DH)],
                    den_sh.at[pl.ds(row_base + 9 * DH, STRIPE - 9 * DH)])

    plsc.subcore_barrier()

    def start_gather(ci, b):
        pltpu.async_copy(
            xl2_hbm.at[src_v.at[pl.ds(ci * K, K)]], rows_v.at[b], gsem[b])

    def wait_gather(ci, b):
        pltpu.make_async_copy(
            xl2_hbm.at[src_v.at[pl.ds(ci * K, K)]], rows_v.at[b],
            gsem[b]).wait()

    def compute_ee(ci, b, n_groups):
        for g in range(K // 16):
            off = b * K + g * 16
            if g < n_groups:
                s16 = src_v[pl.ds(ci * K + g * 16, 16)] - off16
                d16 = dst_v[ci, pl.ds(g * 16, 16)]
                asg = plsc.load_gather(as_v, [s16])
                adg = plsc.load_gather(ad_v, [d16])
                v = asg + adg
                e = jnp.maximum(v, 0.2 * v)
                ee_v[pl.ds(off, 16)] = jnp.exp(e - c_v[...])
            else:
                ee_v[pl.ds(off, 16)] = zero16

    def scale_rows(b):
        @pl.loop(0, K, step=4)
        def _(k0):
            for dk in range(4):
                k = k0 + dk
                esc = plsc.load_gather(
                    ee_v, [jnp.full((16,), b * K, jnp.int32) + k])
                for m in range(DH // 16):
                    sl = pl.ds(m * 16, 16)
                    rows_v[b, k, sl] = rows_v[b, k, sl] * esc

    def start_scatters(ci, b):
        pltpu.async_copy(rows_v.at[b], acc_sh.at[dst_v.at[ci]], rsem[b],
                         add=True)
        pltpu.async_copy(ee_v.at[pl.ds(b * K, K)], den_sh.at[dst_v.at[ci]],
                         esem[b], add=True)

    def wait_scatters(ci_prev, b):
        pltpu.make_async_copy(rows_v.at[b], acc_sh.at[dst_v.at[ci_prev]],
                              rsem[b]).wait()
        pltpu.make_async_copy(ee_v.at[pl.ds(b * K, K)],
                              den_sh.at[dst_v.at[ci_prev]], esem[b]).wait()

    def warm(ci, b, n_groups):
        start_gather(ci, b)
        compute_ee(ci, b, n_groups)

    def fire(ci, b):
        wait_gather(ci, b)
        scale_rows(b)
        start_scatters(ci, b)

    def substep(t, first_warm=False, n_groups=K // 16):
        if t >= 2:
            fire(t - 2, (t - 2) % NB)
        if t < NCH:
            if not first_warm:
                wait_scatters(t - NB, t % NB)
            warm(t, t % NB, n_groups)

    substep(0, first_warm=True)
    substep(1, first_warm=True)
    substep(2, first_warm=True)

    @pl.loop(3, 156, step=3)
    def _(base):
        fire(base - 2, 1)
        wait_scatters(base - 3, 0)
        warm(base, 0, K // 16)
        fire(base - 1, 2)
        wait_scatters(base - 2, 1)
        warm(base + 1, 1, K // 16)
        fire(base, 0)
        wait_scatters(base - 1, 2)
        warm(base + 2, 2, K // 16)

    substep(156, n_groups=VALID_LAST // 16)
    substep(157)
    substep(158)

    wait_scatters(NCH - 3, (NCH - 3) % NB)
    wait_scatters(NCH - 2, (NCH - 2) % NB)
    wait_scatters(NCH - 1, (NCH - 1) % NB)

    plsc.subcore_barrier()

    pltpu.sync_copy(acc_sh.at[pl.ds(row_base, STRIPE)],
                    acc_out.at[pl.ds(row_base, STRIPE),
                               pl.ds(cid * DH, DH)])

    @pl.when(sid == 0)
    def _():
        pltpu.sync_copy(den_sh, den_out.at[cid])


def kernel(x, edge_index, W1, a_src1, a_dst1, b1, W2, a_src2, a_dst2, b2):
    ei = edge_index.astype(jnp.int32)
    src = ei[0].reshape(NS, EPW)
    dst = ei[1].reshape(NS, EPW)
    npad = EPW_PAD - EPW
    pad_s = (jnp.arange(NS * npad, dtype=jnp.int32) * 97 + 13) % N
    pad_d = (jnp.arange(NS * npad, dtype=jnp.int32) * 131 + 7) % N
    src_flat = jnp.concatenate([src, pad_s.reshape(NS, npad)], axis=1)
    dst_chunk = jnp.concatenate([dst, pad_d.reshape(NS, npad)],
                                axis=1).reshape(NS, NCH, K)
    av1 = jnp.stack([a_src1, a_dst1], axis=0)
    av2 = jnp.stack([a_src2, a_dst2], axis=0)

    xlh1, as1, ad1, c1 = _pre(x, W1, av1)
    acc1, den1 = _sc_edges(xlh1.reshape(NC * N, DH), src_flat, dst_chunk,
                           as1.reshape(N), ad1.reshape(N), c1.reshape(16))
    xlh2, as2, ad2, c2 = _mid(acc1, den1[0, :N].reshape(N, 1),
                              as1, ad1, c1, xlh1, b1.reshape(1, D),
                              W2, av2)
    acc2, den2 = _sc_edges(xlh2.reshape(NC * N, DH), src_flat, dst_chunk,
                           as2.reshape(N), ad2.reshape(N), c2.reshape(16))
    return _eps(acc2, den2[0, :N].reshape(N, 1),
                as2, ad2, c2, xlh2, b2.reshape(1, D))
